# Initial kernel scaffold; baseline (speedup 1.0000x reference)
import jax
import jax.numpy as jnp
from jax.experimental import pallas as pl


def kernel(x, adj, W1, b1, W2, b2, W3, b3, W4, b4):
    raise NotImplementedError("write your pallas kernel here")


# speedup vs baseline: 13.9311x; 13.9311x over previous
import functools

import jax
import jax.numpy as jnp
from jax import lax
from jax.experimental import pallas as pl
from jax.experimental.pallas import tpu as pltpu
from jax.experimental.pallas import tpu_sc as plsc

N = 10000
E = 320000
NPAD = 10112
CH = 128
NWORKERS = 32
EPT = 10240
TOT_E = EPT * NWORKERS
NCHUNK = EPT // CH
RPT = NPAD // 16

_mesh = plsc.VectorSubcoreMesh(core_axis_name="c", subcore_axis_name="s")


def _wid():
    return lax.axis_index("c") * 16 + lax.axis_index("s")


@functools.partial(
    pl.kernel,
    mesh=_mesh,
    out_type=jax.ShapeDtypeStruct((2 * NPAD,), jnp.float32),
    compiler_params=pltpu.CompilerParams(use_tc_tiling_on_sc=False),
    scratch_types=[
        pltpu.VMEM((CH,), jnp.int32),
        pltpu.VMEM((CH,), jnp.float32),
        pltpu.VMEM((RPT,), jnp.float32),
        pltpu.VMEM_SHARED((NPAD,), jnp.float32),
    ],
)
def _sc_degree(dst_hbm, out_hbm, didx_v, ones_v, stg_v, acc_sh):
    cid = lax.axis_index("c")
    sid = lax.axis_index("s")
    wid = cid * 16 + sid

    for i in range(CH // 16):
        ones_v[pl.ds(i * 16, 16)] = jnp.ones((16,), jnp.float32)

    def zfill(i, carry):
        stg_v[pl.ds(i * 16, 16)] = jnp.zeros((16,), jnp.float32)
        return carry

    lax.fori_loop(0, RPT // 16, zfill, 0)
    stg_v[pl.ds(RPT - 16, 16)] = jnp.zeros((16,), jnp.float32)
    pltpu.sync_copy(stg_v, acc_sh.at[pl.ds(sid * RPT, RPT)])
    plsc.subcore_barrier()

    def step(c, carry):
        off = wid * EPT + c * CH
        pltpu.sync_copy(dst_hbm.at[pl.ds(off, CH)], didx_v)
        pltpu.sync_copy(ones_v, acc_sh.at[didx_v], add=True)
        return carry

    lax.fori_loop(0, NCHUNK, step, 0)
    plsc.subcore_barrier()
    pltpu.sync_copy(acc_sh.at[pl.ds(sid * RPT, RPT)], stg_v)
    pltpu.sync_copy(stg_v, out_hbm.at[pl.ds(cid * NPAD + sid * RPT, RPT)])


def _make_sc_agg(F):
    @functools.partial(
        pl.kernel,
        mesh=_mesh,
        out_type=jax.ShapeDtypeStruct((2 * NPAD, F), jnp.float32),
        compiler_params=pltpu.CompilerParams(use_tc_tiling_on_sc=False),
        scratch_types=[
            pltpu.VMEM((CH,), jnp.int32),
            pltpu.VMEM((CH,), jnp.int32),
            pltpu.VMEM((CH, F), jnp.float32),
            pltpu.VMEM((RPT, F), jnp.float32),
            pltpu.VMEM_SHARED((NPAD, F), jnp.float32),
            pltpu.SemaphoreType.DMA,
        ],
    )
    def _sc_agg(zs_hbm, src_hbm, dst_hbm, zeros_hbm, out_hbm, sidx_v, didx_v,
                rows_v, stg_v, acc_sh, sem):
        cid = lax.axis_index("c")
        sid = lax.axis_index("s")
        wid = cid * 16 + sid

        pltpu.sync_copy(zeros_hbm.at[pl.ds(sid * RPT, RPT)], stg_v)
        pltpu.sync_copy(stg_v, acc_sh.at[pl.ds(sid * RPT, RPT)])
        plsc.subcore_barrier()

        def step(c, carry):
            off = wid * EPT + c * CH
            pltpu.sync_copy(src_hbm.at[pl.ds(off, CH)], sidx_v)
            pltpu.sync_copy(dst_hbm.at[pl.ds(off, CH)], didx_v)
            pltpu.async_copy(zs_hbm.at[sidx_v], rows_v, sem).wait()
            pltpu.sync_copy(rows_v, acc_sh.at[didx_v], add=True)
            return carry

        lax.fori_loop(0, NCHUNK, step, 0)
        plsc.subcore_barrier()
        pltpu.sync_copy(acc_sh.at[pl.ds(sid * RPT, RPT)], stg_v)
        pltpu.sync_copy(stg_v, out_hbm.at[pl.ds(cid * NPAD + sid * RPT, RPT)])

    return _sc_agg


_sc_agg_8 = _make_sc_agg(8)
_sc_agg_16 = _make_sc_agg(16)
_sc_agg_40 = _make_sc_agg(40)


def _tc0_body(d0_ref, d1_ref, x_ref, w_ref, dinv_ref, zs_ref):
    deg = d0_ref[...] + d1_ref[...] + 1.0
    dinv = lax.rsqrt(deg)
    dinv_ref[...] = dinv
    z = jnp.dot(x_ref[...], w_ref[...], preferred_element_type=jnp.float32)
    zs_ref[...] = z * dinv


def _tc_mid_body(s0_ref, s1_ref, zs_ref, dinv_ref, b_ref, w_ref, zsn_ref):
    o = dinv_ref[...] * (s0_ref[...] + s1_ref[...] + zs_ref[...]) + b_ref[...]
    h = jnp.maximum(o, 0.0)
    z = jnp.dot(h, w_ref[...], preferred_element_type=jnp.float32)
    zsn_ref[...] = z * dinv_ref[...]


def _tc_final_body(s0_ref, s1_ref, zs_ref, dinv_ref, b_ref, out_ref):
    o = dinv_ref[...] * (s0_ref[...] + s1_ref[...] + zs_ref[...]) + b_ref[...]
    m = jnp.max(o, axis=1, keepdims=True)
    e = jnp.exp(o - m)
    out_ref[...] = (o - m) - jnp.log(jnp.sum(e, axis=1, keepdims=True))


def _tc(body, out_shape, *args):
    return pl.pallas_call(body, out_shape=out_shape)(*args)


def kernel(x, adj, W1, b1, W2, b2, W3, b3, W4, b4):
    src = adj[0].astype(jnp.int32)
    dst = adj[1].astype(jnp.int32)
    pad = TOT_E - E
    srcp = jnp.concatenate([src, jnp.zeros((pad,), jnp.int32)])
    dstp = jnp.concatenate([dst, jnp.full((pad,), N, jnp.int32)])

    degp = _sc_degree(dstp)
    d0 = degp[:N]
    d1 = degp[NPAD:NPAD + N]

    f32 = jnp.float32
    dinv, zs = _tc(
        _tc0_body,
        (jax.ShapeDtypeStruct((N, 1), f32), jax.ShapeDtypeStruct((N, 8), f32)),
        d0[:, None], d1[:, None], x, W1,
    )

    zmat = {f: jnp.zeros((NPAD, f), jnp.float32) for f in (8, 16, 40)}
    aggs = (_sc_agg_8, _sc_agg_16, _sc_agg_8)
    fins = (8, 16, 8)
    outs = (16, 8, 40)
    for agg, fi, b, w, fo in zip(aggs, fins, (b1, b2, b3), (W2, W3, W4), outs):
        s = agg(zs, srcp, dstp, zmat[fi])
        zs = _tc(
            _tc_mid_body,
            jax.ShapeDtypeStruct((N, fo), f32),
            s[:N], s[NPAD:NPAD + N], zs, dinv, b[None, :], w,
        )

    s = _sc_agg_40(zs, srcp, dstp, zmat[40])
    return _tc(
        _tc_final_body,
        jax.ShapeDtypeStruct((N, 40), f32),
        s[:N], s[NPAD:NPAD + N], zs, dinv, b4[None, :],
    )


# speedup vs baseline: 26.0441x; 1.8695x over previous
import functools

import jax
import jax.numpy as jnp
from jax import lax
from jax.experimental import pallas as pl
from jax.experimental.pallas import tpu as pltpu
from jax.experimental.pallas import tpu_sc as plsc

N = 10000
E = 320000
NPAD = 10112
CH = 128
NWORKERS = 32
EPT = 10240
TOT_E = EPT * NWORKERS
NCHUNK = EPT // CH
RPT = NPAD // 16

_mesh = plsc.VectorSubcoreMesh(core_axis_name="c", subcore_axis_name="s")


def _wid():
    return lax.axis_index("c") * 16 + lax.axis_index("s")


@functools.partial(
    pl.kernel,
    mesh=_mesh,
    out_type=jax.ShapeDtypeStruct((2 * NPAD,), jnp.float32),
    compiler_params=pltpu.CompilerParams(use_tc_tiling_on_sc=False),
    scratch_types=[
        pltpu.VMEM((NCHUNK, CH), jnp.int32),
        pltpu.VMEM((CH,), jnp.float32),
        pltpu.VMEM((RPT,), jnp.float32),
        pltpu.VMEM_SHARED((NPAD,), jnp.float32),
    ],
)
def _sc_degree(dst_hbm, out_hbm, didx_v, ones_v, stg_v, acc_sh):
    cid = lax.axis_index("c")
    sid = lax.axis_index("s")
    wid = cid * 16 + sid

    pltpu.sync_copy(dst_hbm.at[wid], didx_v)
    for i in range(CH // 16):
        ones_v[pl.ds(i * 16, 16)] = jnp.ones((16,), jnp.float32)

    def zfill(i, carry):
        stg_v[pl.ds(i * 16, 16)] = jnp.zeros((16,), jnp.float32)
        return carry

    lax.fori_loop(0, RPT // 16, zfill, 0)
    stg_v[pl.ds(RPT - 16, 16)] = jnp.zeros((16,), jnp.float32)
    pltpu.sync_copy(stg_v, acc_sh.at[pl.ds(sid * RPT, RPT)])
    plsc.subcore_barrier()

    def step(c, carry):
        pltpu.sync_copy(ones_v, acc_sh.at[didx_v.at[c]], add=True)
        return carry

    lax.fori_loop(0, NCHUNK, step, 0)
    plsc.subcore_barrier()
    pltpu.sync_copy(acc_sh.at[pl.ds(sid * RPT, RPT)], stg_v)
    pltpu.sync_copy(stg_v, out_hbm.at[pl.ds(cid * NPAD + sid * RPT, RPT)])


NBUF = 4


def _make_sc_agg(F):
    @functools.partial(
        pl.kernel,
        mesh=_mesh,
        out_type=jax.ShapeDtypeStruct((2 * NPAD, F), jnp.float32),
        compiler_params=pltpu.CompilerParams(use_tc_tiling_on_sc=False),
        scratch_types=[
            pltpu.VMEM((NCHUNK, CH), jnp.int32),
            pltpu.VMEM((NCHUNK, CH), jnp.int32),
            [pltpu.VMEM((CH, F), jnp.float32) for _ in range(NBUF)],
            [pltpu.SemaphoreType.DMA for _ in range(NBUF)],
            pltpu.VMEM((RPT, F), jnp.float32),
            pltpu.VMEM_SHARED((NPAD, F), jnp.float32),
        ],
    )
    def _sc_agg(zs_hbm, src_hbm, dst_hbm, zeros_hbm, out_hbm, sidx_v, didx_v,
                rows, sems, stg_v, acc_sh):
        cid = lax.axis_index("c")
        sid = lax.axis_index("s")
        wid = cid * 16 + sid

        pltpu.sync_copy(src_hbm.at[wid], sidx_v)
        pltpu.sync_copy(dst_hbm.at[wid], didx_v)
        pltpu.sync_copy(zeros_hbm.at[pl.ds(sid * RPT, RPT)], stg_v)
        pltpu.sync_copy(stg_v, acc_sh.at[pl.ds(sid * RPT, RPT)])
        plsc.subcore_barrier()

        for b in range(NBUF):
            pltpu.async_copy(zs_hbm.at[sidx_v.at[b]], rows[b], sems[b])

        def group(g, carry):
            for b in range(NBUF):
                c = g * NBUF + b
                pltpu.make_async_copy(zs_hbm.at[sidx_v.at[c]], rows[b],
                                      sems[b]).wait()
                pltpu.sync_copy(rows[b], acc_sh.at[didx_v.at[c]], add=True)

                @pl.when(c + NBUF < NCHUNK)
                def _():
                    pltpu.async_copy(zs_hbm.at[sidx_v.at[c + NBUF]], rows[b],
                                     sems[b])

            return carry

        lax.fori_loop(0, NCHUNK // NBUF, group, 0)
        plsc.subcore_barrier()
        pltpu.sync_copy(acc_sh.at[pl.ds(sid * RPT, RPT)], stg_v)
        pltpu.sync_copy(stg_v, out_hbm.at[pl.ds(cid * NPAD + sid * RPT, RPT)])

    return _sc_agg


_sc_agg_8 = _make_sc_agg(8)
_sc_agg_16 = _make_sc_agg(16)
_sc_agg_40 = _make_sc_agg(40)


def _tc0_body(d0_ref, d1_ref, x_ref, w_ref, dinv_ref, zs_ref):
    deg = d0_ref[...] + d1_ref[...] + 1.0
    dinv = lax.rsqrt(deg)
    dinv_ref[...] = dinv
    z = jnp.dot(x_ref[...], w_ref[...], preferred_element_type=jnp.float32)
    zs_ref[...] = z * dinv


def _tc_mid_body(s0_ref, s1_ref, zs_ref, dinv_ref, b_ref, w_ref, zsn_ref):
    o = dinv_ref[...] * (s0_ref[...] + s1_ref[...] + zs_ref[...]) + b_ref[...]
    h = jnp.maximum(o, 0.0)
    z = jnp.dot(h, w_ref[...], preferred_element_type=jnp.float32)
    zsn_ref[...] = z * dinv_ref[...]


def _tc_final_body(s0_ref, s1_ref, zs_ref, dinv_ref, b_ref, out_ref):
    o = dinv_ref[...] * (s0_ref[...] + s1_ref[...] + zs_ref[...]) + b_ref[...]
    m = jnp.max(o, axis=1, keepdims=True)
    e = jnp.exp(o - m)
    out_ref[...] = (o - m) - jnp.log(jnp.sum(e, axis=1, keepdims=True))


def _tc(body, out_shape, *args):
    return pl.pallas_call(body, out_shape=out_shape)(*args)


def kernel(x, adj, W1, b1, W2, b2, W3, b3, W4, b4):
    src = adj[0].astype(jnp.int32)
    dst = adj[1].astype(jnp.int32)
    pad = TOT_E - E
    srcp = jnp.concatenate([src, jnp.zeros((pad,), jnp.int32)])
    dstp = jnp.concatenate([dst, jnp.full((pad,), N, jnp.int32)])
    srcp = srcp.reshape(NWORKERS, NCHUNK, CH)
    dstp = dstp.reshape(NWORKERS, NCHUNK, CH)

    degp = _sc_degree(dstp)
    d0 = degp[:N]
    d1 = degp[NPAD:NPAD + N]

    f32 = jnp.float32
    dinv, zs = _tc(
        _tc0_body,
        (jax.ShapeDtypeStruct((N, 1), f32), jax.ShapeDtypeStruct((N, 8), f32)),
        d0[:, None], d1[:, None], x, W1,
    )

    zmat = {f: jnp.zeros((NPAD, f), jnp.float32) for f in (8, 16, 40)}
    aggs = (_sc_agg_8, _sc_agg_16, _sc_agg_8)
    fins = (8, 16, 8)
    outs = (16, 8, 40)
    for agg, fi, b, w, fo in zip(aggs, fins, (b1, b2, b3), (W2, W3, W4), outs):
        s = agg(zs, srcp, dstp, zmat[fi])
        zs = _tc(
            _tc_mid_body,
            jax.ShapeDtypeStruct((N, fo), f32),
            s[:N], s[NPAD:NPAD + N], zs, dinv, b[None, :], w,
        )

    s = _sc_agg_40(zs, srcp, dstp, zmat[40])
    return _tc(
        _tc_final_body,
        jax.ShapeDtypeStruct((N, 40), f32),
        s[:N], s[NPAD:NPAD + N], zs, dinv, b4[None, :],
    )


# speedup vs baseline: 42.5293x; 1.6330x over previous
import functools

import jax
import jax.numpy as jnp
from jax import lax
from jax.experimental import pallas as pl
from jax.experimental.pallas import tpu as pltpu
from jax.experimental.pallas import tpu_sc as plsc

N = 10000
E = 320000
NPAD = 10112
CH = 128
NWORKERS = 32
EPT = 10240
TOT_E = EPT * NWORKERS
NCHUNK = EPT // CH
RPT = NPAD // 16

_mesh = plsc.VectorSubcoreMesh(core_axis_name="c", subcore_axis_name="s")


def _wid():
    return lax.axis_index("c") * 16 + lax.axis_index("s")


@functools.partial(
    pl.kernel,
    mesh=_mesh,
    out_type=jax.ShapeDtypeStruct((2 * NPAD,), jnp.float32),
    compiler_params=pltpu.CompilerParams(use_tc_tiling_on_sc=False),
    scratch_types=[
        pltpu.VMEM((NCHUNK, CH), jnp.int32),
        pltpu.VMEM((CH,), jnp.float32),
        pltpu.VMEM((RPT,), jnp.float32),
        pltpu.VMEM_SHARED((NPAD,), jnp.float32),
    ],
)
def _sc_degree(dst_hbm, out_hbm, didx_v, ones_v, stg_v, acc_sh):
    cid = lax.axis_index("c")
    sid = lax.axis_index("s")
    wid = cid * 16 + sid

    pltpu.sync_copy(dst_hbm.at[wid], didx_v)
    for i in range(CH // 16):
        ones_v[pl.ds(i * 16, 16)] = jnp.ones((16,), jnp.float32)

    def zfill(i, carry):
        stg_v[pl.ds(i * 16, 16)] = jnp.zeros((16,), jnp.float32)
        return carry

    lax.fori_loop(0, RPT // 16, zfill, 0)
    stg_v[pl.ds(RPT - 16, 16)] = jnp.zeros((16,), jnp.float32)
    pltpu.sync_copy(stg_v, acc_sh.at[pl.ds(sid * RPT, RPT)])
    plsc.subcore_barrier()

    def step(c, carry):
        pltpu.sync_copy(ones_v, acc_sh.at[didx_v.at[c]], add=True)
        return carry

    lax.fori_loop(0, NCHUNK, step, 0)
    plsc.subcore_barrier()
    pltpu.sync_copy(acc_sh.at[pl.ds(sid * RPT, RPT)], stg_v)
    pltpu.sync_copy(stg_v, out_hbm.at[pl.ds(cid * NPAD + sid * RPT, RPT)])


NBUF = 4


def _make_sc_agg(F):
    @functools.partial(
        pl.kernel,
        mesh=_mesh,
        out_type=jax.ShapeDtypeStruct((2 * NPAD, F), jnp.float32),
        compiler_params=pltpu.CompilerParams(use_tc_tiling_on_sc=False),
        scratch_types=[
            pltpu.VMEM((NCHUNK, CH), jnp.int32),
            pltpu.VMEM((NCHUNK, CH), jnp.int32),
            [pltpu.VMEM((CH, F), jnp.float32) for _ in range(NBUF)],
            [pltpu.SemaphoreType.DMA for _ in range(NBUF)],
            pltpu.VMEM((RPT, F), jnp.float32),
            pltpu.VMEM_SHARED((NPAD, F), jnp.float32),
            pltpu.VMEM_SHARED((N, F), jnp.float32),
        ],
    )
    def _sc_agg(zs_hbm, src_hbm, dst_hbm, zeros_hbm, out_hbm, sidx_v, didx_v,
                rows, sems, stg_v, acc_sh, zs_sh):
        cid = lax.axis_index("c")
        sid = lax.axis_index("s")
        wid = cid * 16 + sid

        pltpu.sync_copy(src_hbm.at[wid], sidx_v)
        pltpu.sync_copy(dst_hbm.at[wid], didx_v)
        nst = N // 16
        pltpu.sync_copy(zs_hbm.at[pl.ds(sid * nst, nst)],
                        stg_v.at[pl.ds(0, nst)])
        pltpu.sync_copy(stg_v.at[pl.ds(0, nst)],
                        zs_sh.at[pl.ds(sid * nst, nst)])
        pltpu.sync_copy(zeros_hbm.at[pl.ds(sid * RPT, RPT)], stg_v)
        pltpu.sync_copy(stg_v, acc_sh.at[pl.ds(sid * RPT, RPT)])
        plsc.subcore_barrier()

        for b in range(NBUF):
            pltpu.async_copy(zs_sh.at[sidx_v.at[b]], rows[b], sems[b])

        def group(g, carry):
            for b in range(NBUF):
                c = g * NBUF + b
                pltpu.make_async_copy(zs_sh.at[sidx_v.at[c]], rows[b],
                                      sems[b]).wait()
                pltpu.sync_copy(rows[b], acc_sh.at[didx_v.at[c]], add=True)

                @pl.when(c + NBUF < NCHUNK)
                def _():
                    pltpu.async_copy(zs_sh.at[sidx_v.at[c + NBUF]], rows[b],
                                     sems[b])

            return carry

        lax.fori_loop(0, NCHUNK // NBUF, group, 0)
        plsc.subcore_barrier()
        pltpu.sync_copy(acc_sh.at[pl.ds(sid * RPT, RPT)], stg_v)
        pltpu.sync_copy(stg_v, out_hbm.at[pl.ds(cid * NPAD + sid * RPT, RPT)])

    return _sc_agg


_sc_agg_8 = _make_sc_agg(8)
_sc_agg_16 = _make_sc_agg(16)
_sc_agg_40 = _make_sc_agg(40)


def _tc0_body(d0_ref, d1_ref, x_ref, w_ref, dinv_ref, zs_ref):
    deg = d0_ref[...] + d1_ref[...] + 1.0
    dinv = lax.rsqrt(deg)
    dinv_ref[...] = dinv
    z = jnp.dot(x_ref[...], w_ref[...], preferred_element_type=jnp.float32)
    zs_ref[...] = z * dinv


def _tc_mid_body(s0_ref, s1_ref, zs_ref, dinv_ref, b_ref, w_ref, zsn_ref):
    o = dinv_ref[...] * (s0_ref[...] + s1_ref[...] + zs_ref[...]) + b_ref[...]
    h = jnp.maximum(o, 0.0)
    z = jnp.dot(h, w_ref[...], preferred_element_type=jnp.float32)
    zsn_ref[...] = z * dinv_ref[...]


def _tc_final_body(s0_ref, s1_ref, zs_ref, dinv_ref, b_ref, out_ref):
    o = dinv_ref[...] * (s0_ref[...] + s1_ref[...] + zs_ref[...]) + b_ref[...]
    m = jnp.max(o, axis=1, keepdims=True)
    e = jnp.exp(o - m)
    out_ref[...] = (o - m) - jnp.log(jnp.sum(e, axis=1, keepdims=True))


def _tc(body, out_shape, *args):
    return pl.pallas_call(body, out_shape=out_shape)(*args)


def kernel(x, adj, W1, b1, W2, b2, W3, b3, W4, b4):
    src = adj[0].astype(jnp.int32)
    dst = adj[1].astype(jnp.int32)
    pad = TOT_E - E
    srcp = jnp.concatenate([src, jnp.zeros((pad,), jnp.int32)])
    dstp = jnp.concatenate([dst, jnp.full((pad,), N, jnp.int32)])
    srcp = srcp.reshape(NWORKERS, NCHUNK, CH)
    dstp = dstp.reshape(NWORKERS, NCHUNK, CH)

    degp = _sc_degree(dstp)
    d0 = degp[:N]
    d1 = degp[NPAD:NPAD + N]

    f32 = jnp.float32
    dinv, zs = _tc(
        _tc0_body,
        (jax.ShapeDtypeStruct((N, 1), f32), jax.ShapeDtypeStruct((N, 8), f32)),
        d0[:, None], d1[:, None], x, W1,
    )

    zmat = {f: jnp.zeros((NPAD, f), jnp.float32) for f in (8, 16, 40)}
    aggs = (_sc_agg_8, _sc_agg_16, _sc_agg_8)
    fins = (8, 16, 8)
    outs = (16, 8, 40)
    for agg, fi, b, w, fo in zip(aggs, fins, (b1, b2, b3), (W2, W3, W4), outs):
        s = agg(zs, srcp, dstp, zmat[fi])
        zs = _tc(
            _tc_mid_body,
            jax.ShapeDtypeStruct((N, fo), f32),
            s[:N], s[NPAD:NPAD + N], zs, dinv, b[None, :], w,
        )

    s = _sc_agg_40(zs, srcp, dstp, zmat[40])
    return _tc(
        _tc_final_body,
        jax.ShapeDtypeStruct((N, 40), f32),
        s[:N], s[NPAD:NPAD + N], zs, dinv, b4[None, :],
    )
